# baseline (speedup 1.0000x reference)
import jax
import jax.numpy as jnp
from jax.experimental import pallas as pl
from jax.experimental.pallas import tpu as pltpu


def _layer(h_in, h_prev, wx_ref, wh_ref, b_ref, k, first):
    if first:
        pre = h_in
    else:
        pre = (
            jnp.dot(h_in, wx_ref[k], preferred_element_type=jnp.float32)
            + b_ref[k]
        )
    return jnp.tanh(
        pre + jnp.dot(h_prev, wh_ref[k], preferred_element_type=jnp.float32)
    )


def _rnn_body(seq_ref, x_ref, wx_ref, wh_ref, b_ref, init_ref, *refs,
              seqlen, depth):
    out_refs = refs[:depth]
    B = x_ref.shape[0]
    H = x_ref.shape[2]

    out_refs[0][...] = jax.lax.dot_general(
        x_ref[...], wx_ref[0],
        (((2,), (0,)), ((), ())),
        preferred_element_type=jnp.float32,
    ) + b_ref[0][None]

    h_init = jnp.broadcast_to(init_ref[...], (B, H))
    hs = [h_init] * depth

    for t in range(depth - 1):
        for k in range(t, -1, -1):
            h_in = out_refs[0][:, t, :] if k == 0 else hs[k - 1]
            hs[k] = _layer(h_in, hs[k], wx_ref, wh_ref, b_ref, k, k == 0)
            out_refs[k][:, t - k, :] = hs[k]

    def step(t, hs):
        new = []
        for k in range(depth):
            h_in = out_refs[0][:, t, :] if k == 0 else hs[k - 1]
            new.append(_layer(h_in, hs[k], wx_ref, wh_ref, b_ref, k, k == 0))
        for k in range(depth):
            out_refs[k][:, t - k, :] = new[k]
        return tuple(new)

    hs = jax.lax.fori_loop(depth - 1, seqlen, step, tuple(hs), unroll=8)
    hs = list(hs)

    for t in range(seqlen, seqlen + depth - 1):
        for k in range(depth - 1, t - seqlen, -1):
            hs[k] = _layer(hs[k - 1], hs[k], wx_ref, wh_ref, b_ref, k, False)
            out_refs[k][:, t - k, :] = hs[k]

    t_ids = jax.lax.broadcasted_iota(jnp.int32, (1, seqlen, 1), 1)
    mask = t_ids < seq_ref[...][:, None, :]
    for k in range(depth):
        out_refs[k][...] = jnp.where(mask, out_refs[k][...], 0.0)


def kernel(input, seq_lens, W_x, W_h, b, init_state, batch_size, depth, output_size):
    B, S, H = input.shape
    DEPTH = W_x.shape[0]

    seq2d = seq_lens.reshape(B, 1)
    b3d = b.reshape(DEPTH, 1, H)

    outs = pl.pallas_call(
        lambda *refs: _rnn_body(*refs, seqlen=S, depth=DEPTH),
        grid=(1,),
        in_specs=[
            pl.BlockSpec((B, 1), lambda c: (0, 0)),
            pl.BlockSpec((B, S, H), lambda c: (0, 0, 0)),
            pl.BlockSpec((DEPTH, H, H), lambda c: (0, 0, 0)),
            pl.BlockSpec((DEPTH, H, H), lambda c: (0, 0, 0)),
            pl.BlockSpec((DEPTH, 1, H), lambda c: (0, 0, 0)),
            pl.BlockSpec((1, H), lambda c: (0, 0)),
        ],
        out_specs=tuple(
            pl.BlockSpec((B, S, H), lambda c: (0, 0, 0)) for _ in range(DEPTH)
        ),
        out_shape=tuple(
            jax.ShapeDtypeStruct((B, S, H), jnp.float32) for _ in range(DEPTH)
        ),
    )(seq2d, input, W_x, W_h, b3d, init_state)

    return jnp.stack(outs, axis=2)


# speedup vs baseline: 1.0132x; 1.0132x over previous
import jax
import jax.numpy as jnp
from jax.experimental import pallas as pl
from jax.experimental.pallas import tpu as pltpu


def _rnn_body(seq_ref, x_ref, wx0_ref, wbig_ref, b0_ref, brest_ref, init_ref,
              *out_refs, seqlen, depth):
    B = x_ref.shape[1]
    H = x_ref.shape[2]

    out_refs[0][...] = jax.lax.dot_general(
        x_ref[...], wx0_ref[0],
        (((2,), (0,)), ((), ())),
        preferred_element_type=jnp.float32,
    ) + b0_ref[...][None]

    w_big = wbig_ref[...]
    init = jnp.broadcast_to(init_ref[...], (B, H))
    b_rest = jnp.broadcast_to(brest_ref[...], (B, (depth - 1) * H))

    def fused_step(s, xp):
        add = jnp.concatenate([xp, b_rest], axis=1)
        return jnp.tanh(
            jnp.dot(s, w_big, preferred_element_type=jnp.float32) + add
        )

    s = jnp.concatenate([init] * depth, axis=1)
    for t in range(depth - 1):
        s = fused_step(s, out_refs[0][t])
        parts = [s[:, k * H:(k + 1) * H] for k in range(t + 1)]
        out_refs[0][t] = parts[0]
        for k in range(1, t + 1):
            out_refs[k][t - k] = parts[k]
        s = jnp.concatenate(parts + [init] * (depth - 1 - t), axis=1)

    def step(t, s):
        s = fused_step(s, out_refs[0][t])
        out_refs[0][t] = s[:, :H]
        for k in range(1, depth):
            out_refs[k][t - k] = s[:, k * H:(k + 1) * H]
        return s

    s = jax.lax.fori_loop(depth - 1, seqlen, step, s, unroll=8)

    for t in range(seqlen, seqlen + depth - 1):
        s = fused_step(s, out_refs[0][seqlen - 1])
        for k in range(t - seqlen + 1, depth):
            out_refs[k][t - k] = s[:, k * H:(k + 1) * H]

    t_ids = jax.lax.broadcasted_iota(jnp.int32, (seqlen, 1, 1), 0)
    mask = t_ids < seq_ref[...][None]
    for k in range(depth):
        out_refs[k][...] = jnp.where(mask, out_refs[k][...], 0.0)


def kernel(input, seq_lens, W_x, W_h, b, init_state, batch_size, depth, output_size):
    B, S, H = input.shape
    DEPTH = W_x.shape[0]

    xT = jnp.swapaxes(input, 0, 1)
    seq2d = seq_lens.reshape(B, 1)

    blocks = [
        [jnp.zeros((H, H), jnp.float32) for _ in range(DEPTH)]
        for _ in range(DEPTH)
    ]
    for k in range(DEPTH):
        blocks[k][k] = W_h[k]
        if k + 1 < DEPTH:
            blocks[k][k + 1] = W_x[k + 1]
    w_big = jnp.block(blocks)

    b0 = b[0].reshape(1, H)
    if DEPTH > 1:
        b_rest = b[1:].reshape(1, (DEPTH - 1) * H)
    else:
        b_rest = jnp.zeros((1, H), jnp.float32)

    outs = pl.pallas_call(
        lambda *refs: _rnn_body(*refs, seqlen=S, depth=DEPTH),
        grid=(1,),
        in_specs=[
            pl.BlockSpec((B, 1), lambda c: (0, 0)),
            pl.BlockSpec((S, B, H), lambda c: (0, 0, 0)),
            pl.BlockSpec((1, H, H), lambda c: (0, 0, 0)),
            pl.BlockSpec((DEPTH * H, DEPTH * H), lambda c: (0, 0)),
            pl.BlockSpec((1, H), lambda c: (0, 0)),
            pl.BlockSpec(b_rest.shape, lambda c: (0, 0)),
            pl.BlockSpec((1, H), lambda c: (0, 0)),
        ],
        out_specs=tuple(
            pl.BlockSpec((S, B, H), lambda c: (0, 0, 0)) for _ in range(DEPTH)
        ),
        out_shape=tuple(
            jax.ShapeDtypeStruct((S, B, H), jnp.float32) for _ in range(DEPTH)
        ),
    )(seq2d, xT, W_x[0][None], w_big, b0, b_rest, init_state)

    return jnp.stack([jnp.swapaxes(o, 0, 1) for o in outs], axis=2)
